# baseline (speedup 1.0000x reference)
import functools

import jax
import jax.numpy as jnp
from jax import lax
from jax.experimental import pallas as pl
from jax.experimental.pallas import tpu as pltpu
from jax.experimental.pallas import tpu_sc as plsc

_N = 10000
_E = 320000
_F = 128
_H = 32

_NC = 2
_NS = 16
_NW = _NC * _NS

_SUB = 128
_CH = 1024
_KS = _CH // _SUB
_EW = 10240
_NCH = _EW // _CH
_NR = _EW // _SUB
_EP = _EW * _NW

_mesh = plsc.VectorSubcoreMesh(
    core_axis_name="c", subcore_axis_name="s", num_cores=_NC, num_subcores=_NS)
_sc_params = pltpu.CompilerParams(use_tc_tiling_on_sc=False)
_tc_params = pltpu.CompilerParams(vmem_limit_bytes=100 * 1024 * 1024)

_f32 = jnp.float32


def _deg_phase(src2_hbm, w_hbm, degacc, dall, wbufs, ssems, s):
    def z16(g, carry):
        wbufs[0][pl.ds(pl.multiple_of(g * 16, 16), 16)] = jnp.zeros((16,), _f32)
        return carry
    lax.fori_loop(0, _CH // 16, z16, 0)

    @pl.when(s < 10)
    def _zero():
        pltpu.sync_copy(wbufs[0].at[pl.ds(0, 1000)],
                        degacc.at[pl.ds(s * 1000, 1000)])
    plsc.subcore_barrier()

    for piece in range(2):
        r0 = pl.multiple_of(s * (_NR * 2) + piece * _NR, 8)
        pltpu.sync_copy(src2_hbm.at[pl.ds(r0, _NR)], dall)
        prev = None
        for k in range(_NCH):
            b = k % 2
            e0 = pl.multiple_of(
                s * (_EW * 2) + piece * _EW + k * _CH, _CH)
            pltpu.sync_copy(w_hbm.at[pl.ds(e0, _CH)], wbufs[b])
            ds_ = [pltpu.async_copy(
                wbufs[b].at[pl.ds(j * _SUB, _SUB)],
                degacc.at[dall.at[k * _KS + j]], ssems[b], add=True)
                for j in range(_KS)]
            if prev is not None:
                for d in prev:
                    d.wait()
            prev = ds_
        for d in prev:
            d.wait()
    plsc.subcore_barrier()


def _edge_loop(src2_hbm, w_hbm, vsh, acc, dall, sbufs, wbufs, rowsbufs,
               lsems, gsems, ssems, wid):
    def issue_load(k):
        b = k % 2
        e0 = pl.multiple_of(wid * _EW + k * _CH, _CH)
        r0 = pl.multiple_of(e0 // _SUB, _KS)
        return [pltpu.async_copy(src2_hbm.at[pl.ds(r0, _KS)], sbufs[b],
                                 lsems[b]),
                pltpu.async_copy(w_hbm.at[pl.ds(e0, _CH)], wbufs[b],
                                 lsems[b])]

    def issue_gather(k):
        b = k % 2
        return [pltpu.async_copy(
            vsh.at[sbufs[b].at[j]],
            rowsbufs[b].at[pl.ds(j * _SUB, _SUB)], gsems[b])
            for j in range(_KS)]

    def issue_scatter(k):
        b = k % 2
        return [pltpu.async_copy(
            rowsbufs[b].at[pl.ds(j * _SUB, _SUB)],
            acc.at[dall.at[k * _KS + j]], ssems[b], add=True)
            for j in range(_KS)]

    ld = {0: issue_load(0)}
    for d in ld[0]:
        d.wait()
    gd = {0: issue_gather(0)}
    ld[1] = issue_load(1)
    sd = {}
    for k in range(_NCH):
        b = k % 2
        if k + 1 < _NCH:
            for d in ld[k + 1]:
                d.wait()
            if k >= 1:
                for d in sd[k - 1]:
                    d.wait()
            gd[k + 1] = issue_gather(k + 1)
        for d in gd[k]:
            d.wait()
        rows = rowsbufs[b]
        wall = wbufs[b]

        def scale16(j2, carry, rows=rows, wall=wall):
            off = pl.multiple_of(j2 * 16, 16)
            w16 = wall[pl.ds(off, 16)]
            for e2 in range(16):
                r = off + e2
                bc = jnp.broadcast_to(w16[e2:e2 + 1], (16,))
                rows[r, pl.ds(0, 16)] = rows[r, pl.ds(0, 16)] * bc
                rows[r, pl.ds(16, 16)] = rows[r, pl.ds(16, 16)] * bc
            return carry
        lax.fori_loop(0, _CH // 16, scale16, 0)
        sd[k] = issue_scatter(k)
        if k + 2 < _NCH:
            ld[k + 2] = issue_load(k + 2)
    for d in sd[_NCH - 2]:
        d.wait()
    for d in sd[_NCH - 1]:
        d.wait()


def _newton_rsqrt(deg16):
    y = lax.bitcast_convert_type(
        jnp.int32(0x5F3759DF) - lax.shift_right_logical(
            lax.bitcast_convert_type(deg16, jnp.int32), 1), _f32)
    for _ in range(3):
        y = y * (1.5 - 0.5 * deg16 * y * y)
    return jnp.where(deg16 > 0.0, y, 0.0)


_prop_scratch = [
    pltpu.VMEM_SHARED((_N, _H), _f32),
    pltpu.VMEM_SHARED((_N, _H), _f32),
    pltpu.VMEM((_NR, _SUB), jnp.int32),
    pltpu.VMEM((_KS, _SUB), jnp.int32),
    pltpu.VMEM((_KS, _SUB), jnp.int32),
    pltpu.VMEM((_CH,), _f32),
    pltpu.VMEM((_CH,), _f32),
    pltpu.VMEM((_CH, _H), _f32),
    pltpu.VMEM((_CH, _H), _f32),
    pltpu.VMEM((_CH,), _f32),
    pltpu.SemaphoreType.DMA,
    pltpu.SemaphoreType.DMA,
    pltpu.SemaphoreType.DMA,
    pltpu.SemaphoreType.DMA,
    pltpu.SemaphoreType.DMA,
    pltpu.SemaphoreType.DMA,
]


@functools.partial(
    pl.kernel,
    out_type=(jax.ShapeDtypeStruct((_NC, _N, _H), _f32),
              jax.ShapeDtypeStruct((_N,), _f32)),
    mesh=_mesh,
    compiler_params=_sc_params,
    scratch_types=_prop_scratch[:2] + [pltpu.VMEM_SHARED((_N,), _f32)]
    + _prop_scratch[2:],
)
def _sc_prop1(src2_hbm, dst2_hbm, w_hbm, h0_hbm, z2_hbm,
              out_hbm, dinv_hbm,
              acc, vsh, degacc, dall, sbuf0, sbuf1, wbuf0, wbuf1,
              rows0, rows1, dvbuf,
              lsem0, lsem1, gsem0, gsem1, ssem0, ssem1):
    c = lax.axis_index("c")
    s = lax.axis_index("s")
    wid = c * _NS + s

    _deg_phase(src2_hbm, w_hbm, degacc, dall, (wbuf0, wbuf1),
               (ssem0, ssem1), s)

    @pl.when(s < 10)
    def _stage():
        pltpu.sync_copy(degacc.at[pl.ds(s * 1000, 1000)],
                        dvbuf.at[pl.ds(0, 1000)])

        def newton16(g, carry):
            i = pl.multiple_of(g * 16, 16)
            dvbuf[pl.ds(i, 16)] = _newton_rsqrt(dvbuf[pl.ds(i, 16)])
            return carry
        lax.fori_loop(0, 63, newton16, 0)

        @pl.when(c == 0)
        def _wdinv():
            pltpu.sync_copy(dvbuf.at[pl.ds(0, 1000)],
                            dinv_hbm.at[pl.ds(s * 1000, 1000)])

        pltpu.sync_copy(h0_hbm.at[pl.ds(s * 1000, 1000)],
                        rows0.at[pl.ds(0, 1000)])

        def scal16(g, carry):
            i = pl.multiple_of(g * 16, 16)
            d16 = dvbuf[pl.ds(i, 16)]
            for e2 in range(16):
                r = i + e2
                bc = jnp.broadcast_to(d16[e2:e2 + 1], (16,))
                rows0[r, pl.ds(0, 16)] = rows0[r, pl.ds(0, 16)] * bc
                rows0[r, pl.ds(16, 16)] = rows0[r, pl.ds(16, 16)] * bc
            return carry
        lax.fori_loop(0, 63, scal16, 0)
        pltpu.sync_copy(rows0.at[pl.ds(0, 1000)], vsh.at[pl.ds(s * 1000, 1000)])

        pltpu.sync_copy(z2_hbm.at[pl.ds(s * 1000, 1000)],
                        rows0.at[pl.ds(0, 1000)])
        pltpu.sync_copy(rows0.at[pl.ds(0, 1000)], acc.at[pl.ds(s * 1000, 1000)])

    pltpu.sync_copy(
        dst2_hbm.at[pl.ds(pl.multiple_of(wid * _NR, 8), _NR)], dall)
    plsc.subcore_barrier()

    _edge_loop(src2_hbm, w_hbm, vsh, acc, dall, (sbuf0, sbuf1),
               (wbuf0, wbuf1), (rows0, rows1), (lsem0, lsem1),
               (gsem0, gsem1), (ssem0, ssem1), wid)
    plsc.subcore_barrier()

    @pl.when(s < 10)
    def _out():
        pltpu.sync_copy(acc.at[pl.ds(s * 1000, 1000)], rows0.at[pl.ds(0, 1000)])
        pltpu.sync_copy(rows0.at[pl.ds(0, 1000)],
                        out_hbm.at[c, pl.ds(pl.multiple_of(s * 1000, 8), 1000)])


@functools.partial(
    pl.kernel,
    out_type=jax.ShapeDtypeStruct((_NC, _N, _H), _f32),
    mesh=_mesh,
    compiler_params=_sc_params,
    scratch_types=_prop_scratch,
)
def _sc_prop2(src2_hbm, dst2_hbm, w_hbm, p_hbm, dinv_hbm, z2_hbm, out_hbm,
              acc, vsh, dall, sbuf0, sbuf1, wbuf0, wbuf1, rows0, rows1, dvbuf,
              lsem0, lsem1, gsem0, gsem1, ssem0, ssem1):
    c = lax.axis_index("c")
    s = lax.axis_index("s")
    wid = c * _NS + s

    @pl.when(s < 10)
    def _stage():
        pltpu.sync_copy(dinv_hbm.at[pl.ds(s * 1000, 1000)],
                        dvbuf.at[pl.ds(0, 1000)])
        pltpu.sync_copy(p_hbm.at[0, pl.ds(s * 1000, 1000)],
                        rows0.at[pl.ds(0, 1000)])
        pltpu.sync_copy(p_hbm.at[1, pl.ds(s * 1000, 1000)],
                        rows1.at[pl.ds(0, 1000)])

        def scal16(g, carry):
            i = pl.multiple_of(g * 16, 16)
            d16 = dvbuf[pl.ds(i, 16)]
            nd16 = 0.0 - d16 * d16
            for e2 in range(16):
                r = i + e2
                bc = jnp.broadcast_to(nd16[e2:e2 + 1], (16,))
                rows0[r, pl.ds(0, 16)] = (
                    rows0[r, pl.ds(0, 16)] + rows1[r, pl.ds(0, 16)]) * bc
                rows0[r, pl.ds(16, 16)] = (
                    rows0[r, pl.ds(16, 16)] + rows1[r, pl.ds(16, 16)]) * bc
            return carry
        lax.fori_loop(0, 63, scal16, 0)
        pltpu.sync_copy(rows0.at[pl.ds(0, 1000)], vsh.at[pl.ds(s * 1000, 1000)])

        pltpu.sync_copy(z2_hbm.at[pl.ds(s * 1000, 1000)],
                        rows0.at[pl.ds(0, 1000)])
        pltpu.sync_copy(rows0.at[pl.ds(0, 1000)], acc.at[pl.ds(s * 1000, 1000)])

    pltpu.sync_copy(
        dst2_hbm.at[pl.ds(pl.multiple_of(wid * _NR, 8), _NR)], dall)
    plsc.subcore_barrier()

    _edge_loop(src2_hbm, w_hbm, vsh, acc, dall, (sbuf0, sbuf1),
               (wbuf0, wbuf1), (rows0, rows1), (lsem0, lsem1),
               (gsem0, gsem1), (ssem0, ssem1), wid)
    plsc.subcore_barrier()

    @pl.when(s < 10)
    def _out():
        pltpu.sync_copy(acc.at[pl.ds(s * 1000, 1000)], rows0.at[pl.ds(0, 1000)])
        pltpu.sync_copy(rows0.at[pl.ds(0, 1000)],
                        out_hbm.at[c, pl.ds(pl.multiple_of(s * 1000, 8), 1000)])


def _tc_body(x_ref, h0_ref, c0_ref, p0_ref, p1_ref, q0_ref, q1_ref, dinv_ref,
             wcat_ref, bcat_ref, th0_ref, th1_ref, th2_ref, fcw_ref, fcb_ref,
             out_ref, hn_ref, cn_ref):
    dinv = dinv_ref[...]
    s1 = dinv * (p0_ref[...] + p1_ref[...])
    s2 = dinv * (q0_ref[...] + q1_ref[...])
    a = (jnp.dot(x_ref[...], wcat_ref[...], preferred_element_type=_f32)
         + bcat_ref[...]
         + jnp.dot(h0_ref[...], th0_ref[...], preferred_element_type=_f32)
         + jnp.dot(s1, th1_ref[...], preferred_element_type=_f32)
         + jnp.dot(s2, th2_ref[...], preferred_element_type=_f32))
    gi = jax.nn.sigmoid(a[:, 0 * _H:1 * _H])
    gf = jax.nn.sigmoid(a[:, 1 * _H:2 * _H])
    gt = jnp.tanh(a[:, 2 * _H:3 * _H])
    go = jax.nn.sigmoid(a[:, 3 * _H:4 * _H])
    cn = gf * c0_ref[...] + gi * gt
    hn = go * jnp.tanh(cn)
    cn_ref[...] = cn
    hn_ref[...] = hn
    out_ref[...] = (jnp.dot(hn, fcw_ref[...], preferred_element_type=_f32)
                    + fcb_ref[...])


def kernel(x, edge_index, edge_weight, h_list, c_list,
           W_i, b_i, th_i, tb_i, W_f, b_f, th_f, tb_f,
           W_c, b_c, th_c, tb_c, W_o, b_o, th_o, tb_o, fc_w, fc_b):
    h0 = h_list[0]
    c0 = c_list[0]
    npad = _EP - _E
    srcp = jnp.concatenate([edge_index[0], jnp.zeros((npad,), jnp.int32)])
    dstp = jnp.concatenate([edge_index[1], jnp.zeros((npad,), jnp.int32)])
    wp = jnp.concatenate([edge_weight, jnp.zeros((npad,), _f32)])
    src2 = srcp.reshape(-1, _SUB)
    dst2 = dstp.reshape(-1, _SUB)

    wcat = jnp.concatenate([W_i, W_f, W_c, W_o], axis=1)
    bcat = jnp.concatenate(
        [b_i[0] + tb_i, b_f[0] + tb_f, b_c[0] + tb_c, b_o[0] + tb_o])[None]
    th0c = jnp.concatenate([th_i[0], th_f[0], th_c[0], th_o[0]], axis=1)
    th1c = jnp.concatenate([th_i[1], th_f[1], th_c[1], th_o[1]], axis=1)
    th2c = jnp.concatenate([th_i[2], th_f[2], th_c[2], th_o[2]], axis=1)
    th0m = th0c - th2c
    th1m = -th1c
    th2m = -2.0 * th2c

    z2 = jnp.zeros((_N, _H), _f32)

    p, dinv = _sc_prop1(src2, dst2, wp, h0, z2)
    q = _sc_prop2(src2, dst2, wp, p, dinv, z2)

    _B = 2000
    row = lambda i: (i, 0)
    fixed = lambda i: (0, 0)
    rspec = lambda w: pl.BlockSpec((_B, w), row)
    fspec = lambda r, w: pl.BlockSpec((r, w), fixed)
    out, hn, cn = pl.pallas_call(
        _tc_body,
        grid=(_N // _B,),
        in_specs=[rspec(_F), rspec(_H), rspec(_H), rspec(_H), rspec(_H),
                  rspec(_H), rspec(_H), rspec(1),
                  fspec(_F, 4 * _H), fspec(1, 4 * _H), fspec(_H, 4 * _H),
                  fspec(_H, 4 * _H), fspec(_H, 4 * _H), fspec(_H, 1),
                  fspec(1, 1)],
        out_specs=(rspec(1), rspec(_H), rspec(_H)),
        out_shape=(jax.ShapeDtypeStruct((_N, 1), _f32),
                   jax.ShapeDtypeStruct((_N, _H), _f32),
                   jax.ShapeDtypeStruct((_N, _H), _f32)),
        compiler_params=_tc_params,
    )(x, h0, c0, p[0], p[1], q[0], q[1], dinv.reshape(_N, 1),
      wcat, bcat, th0m, th1m, th2m, fc_w, fc_b.reshape(1, 1))

    return (out, hn[None], cn[None])


# speedup vs baseline: 1.0463x; 1.0463x over previous
import functools

import jax
import jax.numpy as jnp
from jax import lax
from jax.experimental import pallas as pl
from jax.experimental.pallas import tpu as pltpu
from jax.experimental.pallas import tpu_sc as plsc

_N = 10000
_E = 320000
_F = 128
_H = 32

_NC = 2
_NS = 16
_NW = _NC * _NS

_SUB = 128
_CH = 1024
_KS = _CH // _SUB
_EW = 10240
_NCH = _EW // _CH
_NR = _EW // _SUB
_EP = _EW * _NW

_mesh = plsc.VectorSubcoreMesh(
    core_axis_name="c", subcore_axis_name="s", num_cores=_NC, num_subcores=_NS)
_sc_params = pltpu.CompilerParams(use_tc_tiling_on_sc=False)
_tc_params = pltpu.CompilerParams(vmem_limit_bytes=100 * 1024 * 1024)

_f32 = jnp.float32


@functools.partial(
    pl.kernel,
    out_type=jax.ShapeDtypeStruct((_NC * _N,), _f32),
    mesh=_mesh,
    compiler_params=_sc_params,
    scratch_types=[
        pltpu.VMEM_SHARED((_N,), _f32),
        pltpu.VMEM((_EW,), jnp.int32),
        pltpu.VMEM((_EW,), _f32),
        pltpu.SemaphoreType.DMA,
        pltpu.SemaphoreType.DMA,
    ],
)
def _sc_degree(src_hbm, w_hbm, zn_hbm, out_hbm, acc, sbuf, wbuf, sem0, sem1):
    c = lax.axis_index("c")
    s = lax.axis_index("s")
    wid = c * _NS + s

    @pl.when(s < 10)
    def _zero():
        pltpu.sync_copy(zn_hbm.at[pl.ds(s * 1000, 1000)],
                        wbuf.at[pl.ds(0, 1000)])
        pltpu.sync_copy(wbuf.at[pl.ds(0, 1000)], acc.at[pl.ds(s * 1000, 1000)])

    e0 = pl.multiple_of(wid * _EW, 8)
    pltpu.sync_copy(src_hbm.at[pl.ds(e0, _EW)], sbuf)
    plsc.subcore_barrier()
    pltpu.sync_copy(w_hbm.at[pl.ds(e0, _EW)], wbuf)

    sems = (sem0, sem1)
    groups = []
    for g in range(_NCH):
        o = pl.multiple_of(g * _CH, _CH)
        ds_ = [pltpu.async_copy(
            wbuf.at[pl.ds(o, _CH)], acc.at[sbuf.at[pl.ds(o, _CH)]],
            sems[g % 2], add=True)]
        groups.append(ds_)
        if g >= 1:
            for d in groups[g - 1]:
                d.wait()
    for d in groups[_NCH - 1]:
        d.wait()
    plsc.subcore_barrier()

    @pl.when(s < 10)
    def _out():
        pltpu.sync_copy(acc.at[pl.ds(s * 1000, 1000)], wbuf.at[pl.ds(0, 1000)])
        pltpu.sync_copy(wbuf.at[pl.ds(0, 1000)],
                        out_hbm.at[pl.ds(pl.multiple_of(c * _N + s * 1000, 8),
                                         1000)])


def _edge_loop(src_hbm, w_hbm, vsh, acc, dall, sbufs, wbufs, rowsbufs,
               lsems, gsems, ssems, wid):
    def issue_load(k):
        b = k % 2
        e0 = pl.multiple_of(wid * _EW + k * _CH, _CH)
        return [pltpu.async_copy(src_hbm.at[pl.ds(e0, _CH)], sbufs[b],
                                 lsems[b]),
                pltpu.async_copy(w_hbm.at[pl.ds(e0, _CH)], wbufs[b],
                                 lsems[b])]

    def issue_gather(k):
        b = k % 2
        return [pltpu.async_copy(
            vsh.at[sbufs[b]], rowsbufs[b], gsems[b])]

    def issue_scatter(k):
        b = k % 2
        o = pl.multiple_of(k * _CH, _CH)
        return [pltpu.async_copy(
            rowsbufs[b], acc.at[dall.at[pl.ds(o, _CH)]], ssems[b], add=True)]

    ld = {0: issue_load(0)}
    for d in ld[0]:
        d.wait()
    gd = {0: issue_gather(0)}
    ld[1] = issue_load(1)
    sd = {}
    for k in range(_NCH):
        b = k % 2
        if k + 1 < _NCH:
            for d in ld[k + 1]:
                d.wait()
            if k >= 1:
                for d in sd[k - 1]:
                    d.wait()
            gd[k + 1] = issue_gather(k + 1)
        for d in gd[k]:
            d.wait()
        rows = rowsbufs[b]
        wall = wbufs[b]

        def scale16(j2, carry, rows=rows, wall=wall):
            off = pl.multiple_of(j2 * 16, 16)
            w16 = wall[pl.ds(off, 16)]
            for e2 in range(16):
                r = off + e2
                bc = jnp.broadcast_to(w16[e2:e2 + 1], (16,))
                rows[r, pl.ds(0, 16)] = rows[r, pl.ds(0, 16)] * bc
                rows[r, pl.ds(16, 16)] = rows[r, pl.ds(16, 16)] * bc
            return carry
        lax.fori_loop(0, _CH // 16, scale16, 0)
        sd[k] = issue_scatter(k)
        if k + 2 < _NCH:
            ld[k + 2] = issue_load(k + 2)
    for d in sd[_NCH - 2]:
        d.wait()
    for d in sd[_NCH - 1]:
        d.wait()


def _newton_rsqrt(deg16):
    y = lax.bitcast_convert_type(
        jnp.int32(0x5F3759DF) - lax.shift_right_logical(
            lax.bitcast_convert_type(deg16, jnp.int32), 1), _f32)
    for _ in range(3):
        y = y * (1.5 - 0.5 * deg16 * y * y)
    return jnp.where(deg16 > 0.0, y, 0.0)


_prop_scratch = [
    pltpu.VMEM_SHARED((_N, _H), _f32),
    pltpu.VMEM_SHARED((_N, _H), _f32),
    pltpu.VMEM((_EW,), jnp.int32),
    pltpu.VMEM((_CH,), jnp.int32),
    pltpu.VMEM((_CH,), jnp.int32),
    pltpu.VMEM((_CH,), _f32),
    pltpu.VMEM((_CH,), _f32),
    pltpu.VMEM((_CH, _H), _f32),
    pltpu.VMEM((_CH, _H), _f32),
    pltpu.VMEM((_CH,), _f32),
    pltpu.SemaphoreType.DMA,
    pltpu.SemaphoreType.DMA,
    pltpu.SemaphoreType.DMA,
    pltpu.SemaphoreType.DMA,
    pltpu.SemaphoreType.DMA,
    pltpu.SemaphoreType.DMA,
]


@functools.partial(
    pl.kernel,
    out_type=(jax.ShapeDtypeStruct((_NC, _N, _H), _f32),
              jax.ShapeDtypeStruct((_N,), _f32)),
    mesh=_mesh,
    compiler_params=_sc_params,
    scratch_types=_prop_scratch,
)
def _sc_prop1(src_hbm, dst_hbm, w_hbm, h0_hbm, d_hbm, z2_hbm,
              out_hbm, dinv_hbm,
              acc, vsh, dall, sbuf0, sbuf1, wbuf0, wbuf1, rows0, rows1, dvbuf,
              lsem0, lsem1, gsem0, gsem1, ssem0, ssem1):
    c = lax.axis_index("c")
    s = lax.axis_index("s")
    wid = c * _NS + s

    @pl.when(s < 10)
    def _stage():
        pltpu.sync_copy(d_hbm.at[pl.ds(s * 1000, 1000)],
                        dvbuf.at[pl.ds(0, 1000)])
        pltpu.sync_copy(d_hbm.at[pl.ds(pl.multiple_of(_N + s * 1000, 8), 1000)],
                        wbuf0.at[pl.ds(0, 1000)])

        def newton16(g, carry):
            i = pl.multiple_of(g * 16, 16)
            deg16 = dvbuf[pl.ds(i, 16)] + wbuf0[pl.ds(i, 16)]
            dvbuf[pl.ds(i, 16)] = _newton_rsqrt(deg16)
            return carry
        lax.fori_loop(0, 63, newton16, 0)

        @pl.when(c == 0)
        def _wdinv():
            pltpu.sync_copy(dvbuf.at[pl.ds(0, 1000)],
                            dinv_hbm.at[pl.ds(s * 1000, 1000)])

        pltpu.sync_copy(h0_hbm.at[pl.ds(s * 1000, 1000)],
                        rows0.at[pl.ds(0, 1000)])

        def scal16(g, carry):
            i = pl.multiple_of(g * 16, 16)
            d16 = dvbuf[pl.ds(i, 16)]
            for e2 in range(16):
                r = i + e2
                bc = jnp.broadcast_to(d16[e2:e2 + 1], (16,))
                rows0[r, pl.ds(0, 16)] = rows0[r, pl.ds(0, 16)] * bc
                rows0[r, pl.ds(16, 16)] = rows0[r, pl.ds(16, 16)] * bc
            return carry
        lax.fori_loop(0, 63, scal16, 0)
        pltpu.sync_copy(rows0.at[pl.ds(0, 1000)], vsh.at[pl.ds(s * 1000, 1000)])

        pltpu.sync_copy(z2_hbm.at[pl.ds(s * 1000, 1000)],
                        rows0.at[pl.ds(0, 1000)])
        pltpu.sync_copy(rows0.at[pl.ds(0, 1000)], acc.at[pl.ds(s * 1000, 1000)])

    pltpu.sync_copy(
        dst_hbm.at[pl.ds(pl.multiple_of(wid * _EW, 8), _EW)], dall)
    plsc.subcore_barrier()

    _edge_loop(src_hbm, w_hbm, vsh, acc, dall, (sbuf0, sbuf1),
               (wbuf0, wbuf1), (rows0, rows1), (lsem0, lsem1),
               (gsem0, gsem1), (ssem0, ssem1), wid)
    plsc.subcore_barrier()

    @pl.when(s < 10)
    def _out():
        pltpu.sync_copy(acc.at[pl.ds(s * 1000, 1000)], rows0.at[pl.ds(0, 1000)])
        pltpu.sync_copy(rows0.at[pl.ds(0, 1000)],
                        out_hbm.at[c, pl.ds(pl.multiple_of(s * 1000, 8), 1000)])


@functools.partial(
    pl.kernel,
    out_type=jax.ShapeDtypeStruct((_NC, _N, _H), _f32),
    mesh=_mesh,
    compiler_params=_sc_params,
    scratch_types=_prop_scratch,
)
def _sc_prop2(src_hbm, dst_hbm, w_hbm, p_hbm, dinv_hbm, z2_hbm, out_hbm,
              acc, vsh, dall, sbuf0, sbuf1, wbuf0, wbuf1, rows0, rows1, dvbuf,
              lsem0, lsem1, gsem0, gsem1, ssem0, ssem1):
    c = lax.axis_index("c")
    s = lax.axis_index("s")
    wid = c * _NS + s

    @pl.when(s < 10)
    def _stage():
        pltpu.sync_copy(dinv_hbm.at[pl.ds(s * 1000, 1000)],
                        dvbuf.at[pl.ds(0, 1000)])
        pltpu.sync_copy(p_hbm.at[0, pl.ds(s * 1000, 1000)],
                        rows0.at[pl.ds(0, 1000)])
        pltpu.sync_copy(p_hbm.at[1, pl.ds(s * 1000, 1000)],
                        rows1.at[pl.ds(0, 1000)])

        def scal16(g, carry):
            i = pl.multiple_of(g * 16, 16)
            d16 = dvbuf[pl.ds(i, 16)]
            nd16 = 0.0 - d16 * d16
            for e2 in range(16):
                r = i + e2
                bc = jnp.broadcast_to(nd16[e2:e2 + 1], (16,))
                rows0[r, pl.ds(0, 16)] = (
                    rows0[r, pl.ds(0, 16)] + rows1[r, pl.ds(0, 16)]) * bc
                rows0[r, pl.ds(16, 16)] = (
                    rows0[r, pl.ds(16, 16)] + rows1[r, pl.ds(16, 16)]) * bc
            return carry
        lax.fori_loop(0, 63, scal16, 0)
        pltpu.sync_copy(rows0.at[pl.ds(0, 1000)], vsh.at[pl.ds(s * 1000, 1000)])

        pltpu.sync_copy(z2_hbm.at[pl.ds(s * 1000, 1000)],
                        rows0.at[pl.ds(0, 1000)])
        pltpu.sync_copy(rows0.at[pl.ds(0, 1000)], acc.at[pl.ds(s * 1000, 1000)])

    pltpu.sync_copy(
        dst_hbm.at[pl.ds(pl.multiple_of(wid * _EW, 8), _EW)], dall)
    plsc.subcore_barrier()

    _edge_loop(src_hbm, w_hbm, vsh, acc, dall, (sbuf0, sbuf1),
               (wbuf0, wbuf1), (rows0, rows1), (lsem0, lsem1),
               (gsem0, gsem1), (ssem0, ssem1), wid)
    plsc.subcore_barrier()

    @pl.when(s < 10)
    def _out():
        pltpu.sync_copy(acc.at[pl.ds(s * 1000, 1000)], rows0.at[pl.ds(0, 1000)])
        pltpu.sync_copy(rows0.at[pl.ds(0, 1000)],
                        out_hbm.at[c, pl.ds(pl.multiple_of(s * 1000, 8), 1000)])


def _tc_body(x_ref, h0_ref, c0_ref, p0_ref, p1_ref, q0_ref, q1_ref, dinv_ref,
             wcat_ref, bcat_ref, th0_ref, th1_ref, th2_ref, fcw_ref, fcb_ref,
             out_ref, hn_ref, cn_ref):
    dinv = dinv_ref[...]
    s1 = dinv * (p0_ref[...] + p1_ref[...])
    s2 = dinv * (q0_ref[...] + q1_ref[...])
    a = (jnp.dot(x_ref[...], wcat_ref[...], preferred_element_type=_f32)
         + bcat_ref[...]
         + jnp.dot(h0_ref[...], th0_ref[...], preferred_element_type=_f32)
         + jnp.dot(s1, th1_ref[...], preferred_element_type=_f32)
         + jnp.dot(s2, th2_ref[...], preferred_element_type=_f32))
    gi = jax.nn.sigmoid(a[:, 0 * _H:1 * _H])
    gf = jax.nn.sigmoid(a[:, 1 * _H:2 * _H])
    gt = jnp.tanh(a[:, 2 * _H:3 * _H])
    go = jax.nn.sigmoid(a[:, 3 * _H:4 * _H])
    cn = gf * c0_ref[...] + gi * gt
    hn = go * jnp.tanh(cn)
    cn_ref[...] = cn
    hn_ref[...] = hn
    out_ref[...] = (jnp.dot(hn, fcw_ref[...], preferred_element_type=_f32)
                    + fcb_ref[...])


def kernel(x, edge_index, edge_weight, h_list, c_list,
           W_i, b_i, th_i, tb_i, W_f, b_f, th_f, tb_f,
           W_c, b_c, th_c, tb_c, W_o, b_o, th_o, tb_o, fc_w, fc_b):
    h0 = h_list[0]
    c0 = c_list[0]
    npad = _EP - _E
    srcp = jnp.concatenate([edge_index[0], jnp.zeros((npad,), jnp.int32)])
    dstp = jnp.concatenate([edge_index[1], jnp.zeros((npad,), jnp.int32)])
    wp = jnp.concatenate([edge_weight, jnp.zeros((npad,), _f32)])

    wcat = jnp.concatenate([W_i, W_f, W_c, W_o], axis=1)
    bcat = jnp.concatenate(
        [b_i[0] + tb_i, b_f[0] + tb_f, b_c[0] + tb_c, b_o[0] + tb_o])[None]
    th0c = jnp.concatenate([th_i[0], th_f[0], th_c[0], th_o[0]], axis=1)
    th1c = jnp.concatenate([th_i[1], th_f[1], th_c[1], th_o[1]], axis=1)
    th2c = jnp.concatenate([th_i[2], th_f[2], th_c[2], th_o[2]], axis=1)
    th0m = th0c - th2c
    th1m = -th1c
    th2m = -2.0 * th2c

    zn = jnp.zeros((_N,), _f32)
    z2 = jnp.zeros((_N, _H), _f32)

    d_flat = _sc_degree(srcp, wp, zn)
    p, dinv = _sc_prop1(srcp, dstp, wp, h0, d_flat, z2)
    q = _sc_prop2(srcp, dstp, wp, p, dinv, z2)

    _B = 2000
    row = lambda i: (i, 0)
    fixed = lambda i: (0, 0)
    rspec = lambda w: pl.BlockSpec((_B, w), row)
    fspec = lambda r, w: pl.BlockSpec((r, w), fixed)
    out, hn, cn = pl.pallas_call(
        _tc_body,
        grid=(_N // _B,),
        in_specs=[rspec(_F), rspec(_H), rspec(_H), rspec(_H), rspec(_H),
                  rspec(_H), rspec(_H), rspec(1),
                  fspec(_F, 4 * _H), fspec(1, 4 * _H), fspec(_H, 4 * _H),
                  fspec(_H, 4 * _H), fspec(_H, 4 * _H), fspec(_H, 1),
                  fspec(1, 1)],
        out_specs=(rspec(1), rspec(_H), rspec(_H)),
        out_shape=(jax.ShapeDtypeStruct((_N, 1), _f32),
                   jax.ShapeDtypeStruct((_N, _H), _f32),
                   jax.ShapeDtypeStruct((_N, _H), _f32)),
        compiler_params=_tc_params,
    )(x, h0, c0, p[0], p[1], q[0], q[1], dinv.reshape(_N, 1),
      wcat, bcat, th0m, th1m, th2m, fc_w, fc_b.reshape(1, 1))

    return (out, hn[None], cn[None])


# speedup vs baseline: 1.0959x; 1.0474x over previous
import functools

import jax
import jax.numpy as jnp
from jax import lax
from jax.experimental import pallas as pl
from jax.experimental.pallas import tpu as pltpu
from jax.experimental.pallas import tpu_sc as plsc

_N = 10000
_E = 320000
_F = 128
_H = 32

_NC = 2
_NS = 16
_NW = _NC * _NS

_SUB = 128
_CH = 1024
_KS = _CH // _SUB
_EW = 10240
_NCH = _EW // _CH
_NR = _EW // _SUB
_EP = _EW * _NW

_mesh = plsc.VectorSubcoreMesh(
    core_axis_name="c", subcore_axis_name="s", num_cores=_NC, num_subcores=_NS)
_sc_params = pltpu.CompilerParams(use_tc_tiling_on_sc=False)
_tc_params = pltpu.CompilerParams(vmem_limit_bytes=100 * 1024 * 1024)

_f32 = jnp.float32


@functools.partial(
    pl.kernel,
    out_type=jax.ShapeDtypeStruct((_NC * _N,), _f32),
    mesh=_mesh,
    compiler_params=_sc_params,
    scratch_types=[
        pltpu.VMEM_SHARED((_N,), _f32),
        pltpu.VMEM((_EW,), jnp.int32),
        pltpu.VMEM((_EW,), _f32),
        pltpu.SemaphoreType.DMA,
        pltpu.SemaphoreType.DMA,
    ],
)
def _sc_degree(src_hbm, w_hbm, out_hbm, acc, sbuf, wbuf, sem0, sem1):
    c = lax.axis_index("c")
    s = lax.axis_index("s")
    wid = c * _NS + s

    e0 = pl.multiple_of(wid * _EW, 8)
    ds_src = pltpu.async_copy(src_hbm.at[pl.ds(e0, _EW)], sbuf, sem0)

    @pl.when(s < 10)
    def _zero():
        def zero16(g, carry):
            wbuf[pl.ds(pl.multiple_of(g * 16, 16), 16)] = jnp.zeros((16,), _f32)
            return carry
        lax.fori_loop(0, 63, zero16, 0)
        pltpu.sync_copy(wbuf.at[pl.ds(0, 1000)], acc.at[pl.ds(s * 1000, 1000)])

    ds_src.wait()
    plsc.subcore_barrier()
    pltpu.sync_copy(w_hbm.at[pl.ds(e0, _EW)], wbuf)

    sems = (sem0, sem1)
    groups = []
    for g in range(_NCH):
        o = pl.multiple_of(g * _CH, _CH)
        ds_ = [pltpu.async_copy(
            wbuf.at[pl.ds(o, _CH)], acc.at[sbuf.at[pl.ds(o, _CH)]],
            sems[g % 2], add=True)]
        groups.append(ds_)
        if g >= 1:
            for d in groups[g - 1]:
                d.wait()
    for d in groups[_NCH - 1]:
        d.wait()
    plsc.subcore_barrier()

    @pl.when(s < 10)
    def _out():
        pltpu.sync_copy(acc.at[pl.ds(s * 1000, 1000)], wbuf.at[pl.ds(0, 1000)])
        pltpu.sync_copy(wbuf.at[pl.ds(0, 1000)],
                        out_hbm.at[pl.ds(pl.multiple_of(c * _N + s * 1000, 8),
                                         1000)])


def _edge_loop(src_hbm, w_hbm, vsh, acc, dall, sbufs, wbufs, rowsbufs,
               lsems, gsems, ssems, wid):
    def issue_load(k):
        b = k % 2
        e0 = pl.multiple_of(wid * _EW + k * _CH, _CH)
        return [pltpu.async_copy(src_hbm.at[pl.ds(e0, _CH)], sbufs[b],
                                 lsems[b]),
                pltpu.async_copy(w_hbm.at[pl.ds(e0, _CH)], wbufs[b],
                                 lsems[b])]

    def issue_gather(k):
        b = k % 2
        return [pltpu.async_copy(
            vsh.at[sbufs[b]], rowsbufs[b], gsems[b])]

    def issue_scatter(k):
        b = k % 2
        o = pl.multiple_of(k * _CH, _CH)
        return [pltpu.async_copy(
            rowsbufs[b], acc.at[dall.at[pl.ds(o, _CH)]], ssems[b], add=True)]

    ld = {0: issue_load(0)}
    for d in ld[0]:
        d.wait()
    gd = {0: issue_gather(0)}
    ld[1] = issue_load(1)
    sd = {}
    for k in range(_NCH):
        b = k % 2
        if k + 1 < _NCH:
            for d in ld[k + 1]:
                d.wait()
            if k >= 1:
                for d in sd[k - 1]:
                    d.wait()
            gd[k + 1] = issue_gather(k + 1)
        for d in gd[k]:
            d.wait()
        rows = rowsbufs[b]
        wall = wbufs[b]

        def scale16(j2, carry, rows=rows, wall=wall):
            off = pl.multiple_of(j2 * 16, 16)
            w16 = wall[pl.ds(off, 16)]
            for e2 in range(16):
                r = off + e2
                bc = jnp.broadcast_to(w16[e2:e2 + 1], (16,))
                rows[r, pl.ds(0, 16)] = rows[r, pl.ds(0, 16)] * bc
                rows[r, pl.ds(16, 16)] = rows[r, pl.ds(16, 16)] * bc
            return carry
        lax.fori_loop(0, _CH // 16, scale16, 0)
        sd[k] = issue_scatter(k)
        if k + 2 < _NCH:
            ld[k + 2] = issue_load(k + 2)
    for d in sd[_NCH - 2]:
        d.wait()
    for d in sd[_NCH - 1]:
        d.wait()


def _newton_rsqrt(deg16):
    y = lax.bitcast_convert_type(
        jnp.int32(0x5F3759DF) - lax.shift_right_logical(
            lax.bitcast_convert_type(deg16, jnp.int32), 1), _f32)
    for _ in range(3):
        y = y * (1.5 - 0.5 * deg16 * y * y)
    return jnp.where(deg16 > 0.0, y, 0.0)


_prop_scratch = [
    pltpu.VMEM_SHARED((_N, _H), _f32),
    pltpu.VMEM_SHARED((_N, _H), _f32),
    pltpu.VMEM((_EW,), jnp.int32),
    pltpu.VMEM((_CH,), jnp.int32),
    pltpu.VMEM((_CH,), jnp.int32),
    pltpu.VMEM((_CH,), _f32),
    pltpu.VMEM((_CH,), _f32),
    pltpu.VMEM((_CH, _H), _f32),
    pltpu.VMEM((_CH, _H), _f32),
    pltpu.VMEM((_CH,), _f32),
    pltpu.SemaphoreType.DMA,
    pltpu.SemaphoreType.DMA,
    pltpu.SemaphoreType.DMA,
    pltpu.SemaphoreType.DMA,
    pltpu.SemaphoreType.DMA,
    pltpu.SemaphoreType.DMA,
]


@functools.partial(
    pl.kernel,
    out_type=(jax.ShapeDtypeStruct((_NC, _N, _H), _f32),
              jax.ShapeDtypeStruct((_N,), _f32)),
    mesh=_mesh,
    compiler_params=_sc_params,
    scratch_types=_prop_scratch,
)
def _sc_prop1(src_hbm, dst_hbm, w_hbm, h0_hbm, d_hbm,
              out_hbm, dinv_hbm,
              acc, vsh, dall, sbuf0, sbuf1, wbuf0, wbuf1, rows0, rows1, dvbuf,
              lsem0, lsem1, gsem0, gsem1, ssem0, ssem1):
    c = lax.axis_index("c")
    s = lax.axis_index("s")
    wid = c * _NS + s

    dd = pltpu.async_copy(
        dst_hbm.at[pl.ds(pl.multiple_of(wid * _EW, 8), _EW)], dall, ssem0)

    @pl.when(s < 10)
    def _stage():
        da = pltpu.async_copy(d_hbm.at[pl.ds(s * 1000, 1000)],
                              dvbuf.at[pl.ds(0, 1000)], lsem0)
        db = pltpu.async_copy(
            d_hbm.at[pl.ds(pl.multiple_of(_N + s * 1000, 8), 1000)],
            wbuf0.at[pl.ds(0, 1000)], lsem1)
        dh = pltpu.async_copy(h0_hbm.at[pl.ds(s * 1000, 1000)],
                              rows0.at[pl.ds(0, 1000)], gsem0)

        def zero16(g, carry):
            i = pl.multiple_of(g * 16, 16)
            z = jnp.zeros((16,), _f32)
            for e2 in range(16):
                r = i + e2
                rows1[r, pl.ds(0, 16)] = z
                rows1[r, pl.ds(16, 16)] = z
            return carry
        lax.fori_loop(0, 63, zero16, 0)
        pltpu.sync_copy(rows1.at[pl.ds(0, 1000)], acc.at[pl.ds(s * 1000, 1000)])

        da.wait()
        db.wait()

        def newton16(g, carry):
            i = pl.multiple_of(g * 16, 16)
            deg16 = dvbuf[pl.ds(i, 16)] + wbuf0[pl.ds(i, 16)]
            dvbuf[pl.ds(i, 16)] = _newton_rsqrt(deg16)
            return carry
        lax.fori_loop(0, 63, newton16, 0)

        @pl.when(c == 0)
        def _wdinv():
            pltpu.sync_copy(dvbuf.at[pl.ds(0, 1000)],
                            dinv_hbm.at[pl.ds(s * 1000, 1000)])

        dh.wait()

        def scal16(g, carry):
            i = pl.multiple_of(g * 16, 16)
            d16 = dvbuf[pl.ds(i, 16)]
            for e2 in range(16):
                r = i + e2
                bc = jnp.broadcast_to(d16[e2:e2 + 1], (16,))
                rows0[r, pl.ds(0, 16)] = rows0[r, pl.ds(0, 16)] * bc
                rows0[r, pl.ds(16, 16)] = rows0[r, pl.ds(16, 16)] * bc
            return carry
        lax.fori_loop(0, 63, scal16, 0)
        pltpu.sync_copy(rows0.at[pl.ds(0, 1000)], vsh.at[pl.ds(s * 1000, 1000)])

    dd.wait()
    plsc.subcore_barrier()

    _edge_loop(src_hbm, w_hbm, vsh, acc, dall, (sbuf0, sbuf1),
               (wbuf0, wbuf1), (rows0, rows1), (lsem0, lsem1),
               (gsem0, gsem1), (ssem0, ssem1), wid)
    plsc.subcore_barrier()

    @pl.when(s < 10)
    def _out():
        pltpu.sync_copy(acc.at[pl.ds(s * 1000, 1000)], rows0.at[pl.ds(0, 1000)])
        pltpu.sync_copy(rows0.at[pl.ds(0, 1000)],
                        out_hbm.at[c, pl.ds(pl.multiple_of(s * 1000, 8), 1000)])


@functools.partial(
    pl.kernel,
    out_type=jax.ShapeDtypeStruct((_NC, _N, _H), _f32),
    mesh=_mesh,
    compiler_params=_sc_params,
    scratch_types=_prop_scratch,
)
def _sc_prop2(src_hbm, dst_hbm, w_hbm, p_hbm, dinv_hbm, out_hbm,
              acc, vsh, dall, sbuf0, sbuf1, wbuf0, wbuf1, rows0, rows1, dvbuf,
              lsem0, lsem1, gsem0, gsem1, ssem0, ssem1):
    c = lax.axis_index("c")
    s = lax.axis_index("s")
    wid = c * _NS + s

    dd = pltpu.async_copy(
        dst_hbm.at[pl.ds(pl.multiple_of(wid * _EW, 8), _EW)], dall, ssem0)

    @pl.when(s < 10)
    def _stage():
        da = pltpu.async_copy(dinv_hbm.at[pl.ds(s * 1000, 1000)],
                              dvbuf.at[pl.ds(0, 1000)], lsem0)
        d0 = pltpu.async_copy(p_hbm.at[0, pl.ds(s * 1000, 1000)],
                              rows0.at[pl.ds(0, 1000)], lsem1)
        d1 = pltpu.async_copy(p_hbm.at[1, pl.ds(s * 1000, 1000)],
                              rows1.at[pl.ds(0, 1000)], gsem0)
        da.wait()
        d0.wait()
        d1.wait()

        def scal16(g, carry):
            i = pl.multiple_of(g * 16, 16)
            d16 = dvbuf[pl.ds(i, 16)]
            nd16 = 0.0 - d16 * d16
            for e2 in range(16):
                r = i + e2
                bc = jnp.broadcast_to(nd16[e2:e2 + 1], (16,))
                rows0[r, pl.ds(0, 16)] = (
                    rows0[r, pl.ds(0, 16)] + rows1[r, pl.ds(0, 16)]) * bc
                rows0[r, pl.ds(16, 16)] = (
                    rows0[r, pl.ds(16, 16)] + rows1[r, pl.ds(16, 16)]) * bc
            return carry
        lax.fori_loop(0, 63, scal16, 0)
        pltpu.sync_copy(rows0.at[pl.ds(0, 1000)], vsh.at[pl.ds(s * 1000, 1000)])

        def zero16(g, carry):
            i = pl.multiple_of(g * 16, 16)
            z = jnp.zeros((16,), _f32)
            for e2 in range(16):
                r = i + e2
                rows1[r, pl.ds(0, 16)] = z
                rows1[r, pl.ds(16, 16)] = z
            return carry
        lax.fori_loop(0, 63, zero16, 0)
        pltpu.sync_copy(rows1.at[pl.ds(0, 1000)], acc.at[pl.ds(s * 1000, 1000)])

    dd.wait()
    plsc.subcore_barrier()

    _edge_loop(src_hbm, w_hbm, vsh, acc, dall, (sbuf0, sbuf1),
               (wbuf0, wbuf1), (rows0, rows1), (lsem0, lsem1),
               (gsem0, gsem1), (ssem0, ssem1), wid)
    plsc.subcore_barrier()

    @pl.when(s < 10)
    def _out():
        pltpu.sync_copy(acc.at[pl.ds(s * 1000, 1000)], rows0.at[pl.ds(0, 1000)])
        pltpu.sync_copy(rows0.at[pl.ds(0, 1000)],
                        out_hbm.at[c, pl.ds(pl.multiple_of(s * 1000, 8), 1000)])


def _tc_body(x_ref, h0_ref, c0_ref, p0_ref, p1_ref, q0_ref, q1_ref, dinv_ref,
             wcat_ref, bcat_ref, th0_ref, th1_ref, th2_ref, fcw_ref, fcb_ref,
             out_ref, hn_ref, cn_ref):
    dinv = dinv_ref[...]
    s1 = dinv * (p0_ref[...] + p1_ref[...])
    s2 = dinv * (q0_ref[...] + q1_ref[...])
    a = (jnp.dot(x_ref[...], wcat_ref[...], preferred_element_type=_f32)
         + bcat_ref[...]
         + jnp.dot(h0_ref[...], th0_ref[...], preferred_element_type=_f32)
         + jnp.dot(s1, th1_ref[...], preferred_element_type=_f32)
         + jnp.dot(s2, th2_ref[...], preferred_element_type=_f32))
    gi = jax.nn.sigmoid(a[:, 0 * _H:1 * _H])
    gf = jax.nn.sigmoid(a[:, 1 * _H:2 * _H])
    gt = jnp.tanh(a[:, 2 * _H:3 * _H])
    go = jax.nn.sigmoid(a[:, 3 * _H:4 * _H])
    cn = gf * c0_ref[...] + gi * gt
    hn = go * jnp.tanh(cn)
    cn_ref[...] = cn
    hn_ref[...] = hn
    out_ref[...] = (jnp.dot(hn, fcw_ref[...], preferred_element_type=_f32)
                    + fcb_ref[...])


def kernel(x, edge_index, edge_weight, h_list, c_list,
           W_i, b_i, th_i, tb_i, W_f, b_f, th_f, tb_f,
           W_c, b_c, th_c, tb_c, W_o, b_o, th_o, tb_o, fc_w, fc_b):
    h0 = h_list[0]
    c0 = c_list[0]
    npad = _EP - _E
    srcp = jnp.concatenate([edge_index[0], jnp.zeros((npad,), jnp.int32)])
    dstp = jnp.concatenate([edge_index[1], jnp.zeros((npad,), jnp.int32)])
    wp = jnp.concatenate([edge_weight, jnp.zeros((npad,), _f32)])

    wcat = jnp.concatenate([W_i, W_f, W_c, W_o], axis=1)
    bcat = jnp.concatenate(
        [b_i[0] + tb_i, b_f[0] + tb_f, b_c[0] + tb_c, b_o[0] + tb_o])[None]
    th0c = jnp.concatenate([th_i[0], th_f[0], th_c[0], th_o[0]], axis=1)
    th1c = jnp.concatenate([th_i[1], th_f[1], th_c[1], th_o[1]], axis=1)
    th2c = jnp.concatenate([th_i[2], th_f[2], th_c[2], th_o[2]], axis=1)
    th0m = th0c - th2c
    th1m = -th1c
    th2m = -2.0 * th2c

    d_flat = _sc_degree(srcp, wp)
    p, dinv = _sc_prop1(srcp, dstp, wp, h0, d_flat)
    q = _sc_prop2(srcp, dstp, wp, p, dinv)

    _B = 2000
    row = lambda i: (i, 0)
    fixed = lambda i: (0, 0)
    rspec = lambda w: pl.BlockSpec((_B, w), row)
    fspec = lambda r, w: pl.BlockSpec((r, w), fixed)
    out, hn, cn = pl.pallas_call(
        _tc_body,
        grid=(_N // _B,),
        in_specs=[rspec(_F), rspec(_H), rspec(_H), rspec(_H), rspec(_H),
                  rspec(_H), rspec(_H), rspec(1),
                  fspec(_F, 4 * _H), fspec(1, 4 * _H), fspec(_H, 4 * _H),
                  fspec(_H, 4 * _H), fspec(_H, 4 * _H), fspec(_H, 1),
                  fspec(1, 1)],
        out_specs=(rspec(1), rspec(_H), rspec(_H)),
        out_shape=(jax.ShapeDtypeStruct((_N, 1), _f32),
                   jax.ShapeDtypeStruct((_N, _H), _f32),
                   jax.ShapeDtypeStruct((_N, _H), _f32)),
        compiler_params=_tc_params,
    )(x, h0, c0, p[0], p[1], q[0], q[1], dinv.reshape(_N, 1),
      wcat, bcat, th0m, th1m, th2m, fc_w, fc_b.reshape(1, 1))

    return (out, hn[None], cn[None])
